# baseline (speedup 1.0000x reference)
import functools

import jax
import jax.numpy as jnp
import numpy as np
from jax import lax
from jax.experimental import pallas as pl
from jax.experimental.pallas import tpu as pltpu
from jax.experimental.pallas import tpu_sc as plsc

N = 10000
E = 320000
IN_DIM = 128
H = 8
D = 16
HD = H * D

NC = 2
NS = 16
CH = H // NC
KC = CH * D
EPT = E // NS
C = 80
NCHUNK = EPT // C
ROW = 80
N_PAD = 10240
RPS = N_PAD // NS

INV_SQRT_D = 1.0 / np.sqrt(D)


def _proj_body(h_ref, wq_ref, wkv_ref, bq_ref, bkv_ref, q_ref, kv_ref):
    hb = h_ref[...]
    q_ref[...] = jnp.dot(hb, wq_ref[...], preferred_element_type=jnp.float32) + bq_ref[...]
    kv_ref[...] = jnp.dot(hb, wkv_ref[...], preferred_element_type=jnp.float32) + bkv_ref[...]


def _projections(h, wq, wkv, bq, bkv):
    blk = 400
    grid = N // blk
    return pl.pallas_call(
        _proj_body,
        grid=(grid,),
        in_specs=[
            pl.BlockSpec((blk, IN_DIM), lambda i: (i, 0)),
            pl.BlockSpec((IN_DIM, HD), lambda i: (0, 0)),
            pl.BlockSpec((IN_DIM, 2 * HD), lambda i: (0, 0)),
            pl.BlockSpec((1, HD), lambda i: (0, 0)),
            pl.BlockSpec((1, 2 * HD), lambda i: (0, 0)),
        ],
        out_specs=[
            pl.BlockSpec((blk, HD), lambda i: (i, 0)),
            pl.BlockSpec((blk, 2 * HD), lambda i: (i, 0)),
        ],
        out_shape=[
            jax.ShapeDtypeStruct((N, HD), jnp.float32),
            jax.ShapeDtypeStruct((N, 2 * HD), jnp.float32),
        ],
    )(h, wq, wkv, bq, bkv)


def _sc_body(kv_hbm, q_hbm, src_hbm, dst_hbm, part_hbm,
             src_v, dst0, dst1, gs0, gs1, gd0, gd1, kv0, kv1, q0, q1,
             m_v, zero_v, acc, semg0, semg1):
    c = lax.axis_index("c")
    s = lax.axis_index("s")
    coff = c * N
    dst_b = (dst0, dst1)
    gs_b = (gs0, gs1)
    gd_b = (gd0, gd1)
    kv_b = (kv0, kv1)
    q_b = (q0, q1)
    sem_b = (semg0, semg1)

    def zrow(i, _):
        for j in range(ROW // 16):
            zero_v[i, pl.ds(j * 16, 16)] = jnp.zeros((16,), jnp.float32)
        return 0
    lax.fori_loop(0, 128, zrow, 0)
    for k in range(RPS // 128):
        pltpu.sync_copy(zero_v, acc.at[pl.ds(pl.multiple_of(s * RPS + k * 128, 8), 128)])

    def zpad(e, _):
        m_v[e, pl.ds(KC, 16)] = jnp.zeros((16,), jnp.float32)
        return 0
    lax.fori_loop(0, C, zpad, 0)
    plsc.subcore_barrier()

    lane = lax.broadcasted_iota(jnp.int32, (16,), 0)

    def prefetch(ci, t):
        base = s * EPT + ci * C
        pltpu.sync_copy(src_hbm.at[pl.ds(base, C)], src_v)
        pltpu.sync_copy(dst_hbm.at[pl.ds(base, C)], dst_b[t])
        for j in range(C // 16):
            sl = pl.ds(j * 16, 16)
            gs_b[t][sl] = src_v[sl] + coff
            gd_b[t][sl] = dst_b[t][sl] + coff
        pltpu.async_copy(kv_hbm.at[gs_b[t]], kv_b[t], sem_b[t])
        pltpu.async_copy(q_hbm.at[gd_b[t]], q_b[t], sem_b[t])

    def process(t):
        pltpu.make_async_copy(kv_hbm.at[gs_b[t]], kv_b[t], sem_b[t]).wait()
        pltpu.make_async_copy(q_hbm.at[gd_b[t]], q_b[t], sem_b[t]).wait()
        kv_v = kv_b[t]
        q_v = q_b[t]

        @plsc.parallel_loop(0, C // 16, 1, unroll=2)
        def group(g):
            rows = g * 16 + lane
            for hh in range(CH):
                prods = []
                for d in range(D):
                    col = jnp.full((16,), hh * D + d, jnp.int32)
                    kt = plsc.load_gather(kv_v, [rows, col])
                    qt = plsc.load_gather(q_v, [rows, col])
                    prods.append(kt * qt)
                while len(prods) > 1:
                    prods = [a + b for a, b in zip(prods[::2], prods[1::2])]
                sc = jnp.exp(jnp.clip(prods[0] * INV_SQRT_D, -5.0, 5.0))
                plsc.store_scatter(m_v, [rows, jnp.full((16,), KC + hh, jnp.int32)], sc)
                for d in range(D):
                    vcol = jnp.full((16,), KC + hh * D + d, jnp.int32)
                    vt = plsc.load_gather(kv_v, [rows, vcol])
                    plsc.store_scatter(
                        m_v, [rows, jnp.full((16,), hh * D + d, jnp.int32)], vt * sc)

        pltpu.sync_copy(m_v, acc.at[dst_b[t]], add=True)

    prefetch(0, 0)

    def pair(i2, _):
        prefetch(2 * i2 + 1, 1)
        process(0)
        prefetch(2 * i2 + 2, 0)
        process(1)
        return 0

    lax.fori_loop(0, NCHUNK // 2 - 1, pair, 0)
    prefetch(NCHUNK - 1, 1)
    process(0)
    process(1)
    plsc.subcore_barrier()

    row0 = pl.multiple_of(s * RPS, 8)
    pltpu.sync_copy(acc.at[pl.ds(row0, RPS)], part_hbm.at[c, pl.ds(row0, RPS)])


def _sc_edge_phase(kv2, q2, src, dst):
    mesh = plsc.VectorSubcoreMesh(core_axis_name="c", subcore_axis_name="s",
                                  num_cores=NC, num_subcores=NS)
    f = pl.kernel(
        _sc_body,
        out_type=jax.ShapeDtypeStruct((NC, N_PAD, ROW), jnp.float32),
        mesh=mesh,
        compiler_params=pltpu.CompilerParams(needs_layout_passes=False,
                                             use_tc_tiling_on_sc=False),
        scratch_types=[
            pltpu.VMEM((C,), jnp.int32),
            pltpu.VMEM((C,), jnp.int32),
            pltpu.VMEM((C,), jnp.int32),
            pltpu.VMEM((C,), jnp.int32),
            pltpu.VMEM((C,), jnp.int32),
            pltpu.VMEM((C,), jnp.int32),
            pltpu.VMEM((C,), jnp.int32),
            pltpu.VMEM((C, 2 * KC), jnp.float32),
            pltpu.VMEM((C, 2 * KC), jnp.float32),
            pltpu.VMEM((C, KC), jnp.float32),
            pltpu.VMEM((C, KC), jnp.float32),
            pltpu.VMEM((C, ROW), jnp.float32),
            pltpu.VMEM((128, ROW), jnp.float32),
            pltpu.VMEM_SHARED((N_PAD, ROW), jnp.float32),
            pltpu.SemaphoreType.DMA,
            pltpu.SemaphoreType.DMA,
        ],
    )
    return f(kv2, q2, src, dst)


def _combine_body(p0_ref, p1_ref, s1a_ref, s1b_ref, s2a_ref, s2b_ref, o_ref):
    p0 = p0_ref[0]
    p1 = p1_ref[0]
    wv = (jnp.dot(p0, s1a_ref[...], preferred_element_type=jnp.float32)
          + jnp.dot(p1, s1b_ref[...], preferred_element_type=jnp.float32))
    z = (jnp.dot(p0, s2a_ref[...], preferred_element_type=jnp.float32)
         + jnp.dot(p1, s2b_ref[...], preferred_element_type=jnp.float32))
    o_ref[...] = wv / (z + 1e-6)


def _combine(part, s1a, s1b, s2a, s2b):
    blk = 400
    grid = N // blk
    return pl.pallas_call(
        _combine_body,
        grid=(grid,),
        in_specs=[
            pl.BlockSpec((1, blk, ROW), lambda i: (0, i, 0)),
            pl.BlockSpec((1, blk, ROW), lambda i: (1, i, 0)),
            pl.BlockSpec((ROW, HD), lambda i: (0, 0)),
            pl.BlockSpec((ROW, HD), lambda i: (0, 0)),
            pl.BlockSpec((ROW, HD), lambda i: (0, 0)),
            pl.BlockSpec((ROW, HD), lambda i: (0, 0)),
        ],
        out_specs=pl.BlockSpec((blk, HD), lambda i: (i, 0)),
        out_shape=jax.ShapeDtypeStruct((N, HD), jnp.float32),
    )(part, part, s1a, s1b, s2a, s2b)


def _selectors():
    s1a = np.zeros((ROW, HD), np.float32)
    s1b = np.zeros((ROW, HD), np.float32)
    s2a = np.zeros((ROW, HD), np.float32)
    s2b = np.zeros((ROW, HD), np.float32)
    for hl in range(CH):
        for d in range(D):
            s1a[hl * D + d, hl * D + d] = 1.0
            s1b[hl * D + d, KC + hl * D + d] = 1.0
            s2a[KC + hl, hl * D + d] = 1.0
            s2b[KC + hl, KC + hl * D + d] = 1.0
    return s1a, s1b, s2a, s2b


_S1A, _S1B, _S2A, _S2B = _selectors()


def kernel(h, edge_index, Wq, bq, Wk, bk, Wv, bv):
    wkv = jnp.concatenate([Wk[:, :KC], Wv[:, :KC], Wk[:, KC:], Wv[:, KC:]], axis=1)
    bkv = jnp.concatenate([bk[:KC], bv[:KC], bk[KC:], bv[KC:]])[None, :]
    q, kvp = _projections(h, Wq, wkv, bq[None, :], bkv)
    kv2 = jnp.concatenate([kvp[:, :2 * KC], kvp[:, 2 * KC:]], axis=0)
    q2 = jnp.concatenate([q[:, :KC], q[:, KC:]], axis=0)
    part = _sc_edge_phase(kv2, q2, edge_index[0], edge_index[1])
    return _combine(part, jnp.asarray(_S1A), jnp.asarray(_S1B),
                    jnp.asarray(_S2A), jnp.asarray(_S2B))


# speedup vs baseline: 4.3579x; 4.3579x over previous
import functools

import jax
import jax.numpy as jnp
import numpy as np
from jax import lax
from jax.experimental import pallas as pl
from jax.experimental.pallas import tpu as pltpu
from jax.experimental.pallas import tpu_sc as plsc

N = 10000
E = 320000
IN_DIM = 128
H = 8
D = 16
HD = H * D

NC = 2
NS = 16
CH = H // NC
KC = CH * D
EPT = E // NS
C = 80
NCHUNK = EPT // C
ROW = 80
N_PAD = 10240
RPS = N_PAD // NS

INV_SQRT_D = 1.0 / np.sqrt(D)


def _proj_body(h_ref, wq_ref, wkv_ref, bq_ref, bkv_ref, q_ref, kv_ref):
    hb = h_ref[...]
    q_ref[...] = jnp.dot(hb, wq_ref[...], preferred_element_type=jnp.float32) + bq_ref[...]
    kv_ref[...] = jnp.dot(hb, wkv_ref[...], preferred_element_type=jnp.float32) + bkv_ref[...]


def _projections(h, wq, wkv, bq, bkv):
    blk = 400
    grid = N // blk
    return pl.pallas_call(
        _proj_body,
        grid=(grid,),
        in_specs=[
            pl.BlockSpec((blk, IN_DIM), lambda i: (i, 0)),
            pl.BlockSpec((IN_DIM, HD), lambda i: (0, 0)),
            pl.BlockSpec((IN_DIM, 2 * HD), lambda i: (0, 0)),
            pl.BlockSpec((1, HD), lambda i: (0, 0)),
            pl.BlockSpec((1, 2 * HD), lambda i: (0, 0)),
        ],
        out_specs=[
            pl.BlockSpec((blk, HD), lambda i: (i, 0)),
            pl.BlockSpec((blk, 2 * HD), lambda i: (i, 0)),
        ],
        out_shape=[
            jax.ShapeDtypeStruct((N, HD), jnp.float32),
            jax.ShapeDtypeStruct((N, 2 * HD), jnp.float32),
        ],
    )(h, wq, wkv, bq, bkv)


def _sc_body(kv_hbm, q_hbm, src_hbm, dst_hbm, part_hbm,
             src_v, dst0, dst1, gs0, gs1, gd0, gd1, kv0, kv1, q0, q1,
             m_v, zero_v, acc, semg0, semg1):
    c = lax.axis_index("c")
    s = lax.axis_index("s")
    coff = c * N
    dst_b = (dst0, dst1)
    gs_b = (gs0, gs1)
    gd_b = (gd0, gd1)
    kv_b = (kv0, kv1)
    q_b = (q0, q1)
    sem_b = (semg0, semg1)

    def zrow(i, _):
        for j in range(ROW // 16):
            zero_v[i, pl.ds(j * 16, 16)] = jnp.zeros((16,), jnp.float32)
        return 0
    lax.fori_loop(0, 128, zrow, 0)
    for k in range(RPS // 128):
        pltpu.sync_copy(zero_v, acc.at[pl.ds(pl.multiple_of(s * RPS + k * 128, 8), 128)])

    plsc.subcore_barrier()

    lane = lax.broadcasted_iota(jnp.int32, (16,), 0)

    def prefetch(ci, t):
        base = s * EPT + ci * C
        pltpu.sync_copy(src_hbm.at[pl.ds(base, C)], src_v)
        pltpu.sync_copy(dst_hbm.at[pl.ds(base, C)], dst_b[t])
        for j in range(C // 16):
            sl = pl.ds(j * 16, 16)
            gs_b[t][sl] = src_v[sl] + coff
            gd_b[t][sl] = dst_b[t][sl] + coff
        pltpu.async_copy(kv_hbm.at[gs_b[t]], kv_b[t], sem_b[t])
        pltpu.async_copy(q_hbm.at[gd_b[t]], q_b[t], sem_b[t])

    def process(t):
        pltpu.make_async_copy(kv_hbm.at[gs_b[t]], kv_b[t], sem_b[t]).wait()
        pltpu.make_async_copy(q_hbm.at[gd_b[t]], q_b[t], sem_b[t]).wait()
        kv_v = kv_b[t]
        q_v = q_b[t]

        @plsc.parallel_loop(0, C, 1, unroll=2)
        def edge(e):
            zacc = jnp.zeros((16,), jnp.float32)
            for hh in range(CH):
                kvec = kv_v[e, pl.ds(hh * D, D)]
                qvec = q_v[e, pl.ds(hh * D, D)]
                sca = jnp.sum(kvec * qvec) * INV_SQRT_D
                ex = jnp.exp(jnp.clip(jnp.full((16,), sca), -5.0, 5.0))
                m_v[e, pl.ds(hh * D, D)] = kv_v[e, pl.ds(KC + hh * D, D)] * ex
                zacc = zacc + jnp.where(lane == hh, ex, 0.0)
            m_v[e, pl.ds(KC, 16)] = zacc

        pltpu.sync_copy(m_v, acc.at[dst_b[t]], add=True)

    prefetch(0, 0)

    def pair(i2, _):
        prefetch(2 * i2 + 1, 1)
        process(0)
        prefetch(2 * i2 + 2, 0)
        process(1)
        return 0

    lax.fori_loop(0, NCHUNK // 2 - 1, pair, 0)
    prefetch(NCHUNK - 1, 1)
    process(0)
    process(1)
    plsc.subcore_barrier()

    row0 = pl.multiple_of(s * RPS, 8)
    pltpu.sync_copy(acc.at[pl.ds(row0, RPS)], part_hbm.at[c, pl.ds(row0, RPS)])


def _sc_edge_phase(kv2, q2, src, dst):
    mesh = plsc.VectorSubcoreMesh(core_axis_name="c", subcore_axis_name="s",
                                  num_cores=NC, num_subcores=NS)
    f = pl.kernel(
        _sc_body,
        out_type=jax.ShapeDtypeStruct((NC, N_PAD, ROW), jnp.float32),
        mesh=mesh,
        compiler_params=pltpu.CompilerParams(needs_layout_passes=False,
                                             use_tc_tiling_on_sc=False),
        scratch_types=[
            pltpu.VMEM((C,), jnp.int32),
            pltpu.VMEM((C,), jnp.int32),
            pltpu.VMEM((C,), jnp.int32),
            pltpu.VMEM((C,), jnp.int32),
            pltpu.VMEM((C,), jnp.int32),
            pltpu.VMEM((C,), jnp.int32),
            pltpu.VMEM((C,), jnp.int32),
            pltpu.VMEM((C, 2 * KC), jnp.float32),
            pltpu.VMEM((C, 2 * KC), jnp.float32),
            pltpu.VMEM((C, KC), jnp.float32),
            pltpu.VMEM((C, KC), jnp.float32),
            pltpu.VMEM((C, ROW), jnp.float32),
            pltpu.VMEM((128, ROW), jnp.float32),
            pltpu.VMEM_SHARED((N_PAD, ROW), jnp.float32),
            pltpu.SemaphoreType.DMA,
            pltpu.SemaphoreType.DMA,
        ],
    )
    return f(kv2, q2, src, dst)


def _combine_body(p0_ref, p1_ref, s1a_ref, s1b_ref, s2a_ref, s2b_ref, o_ref):
    p0 = p0_ref[0]
    p1 = p1_ref[0]
    wv = (jnp.dot(p0, s1a_ref[...], preferred_element_type=jnp.float32)
          + jnp.dot(p1, s1b_ref[...], preferred_element_type=jnp.float32))
    z = (jnp.dot(p0, s2a_ref[...], preferred_element_type=jnp.float32)
         + jnp.dot(p1, s2b_ref[...], preferred_element_type=jnp.float32))
    o_ref[...] = wv / (z + 1e-6)


def _combine(part, s1a, s1b, s2a, s2b):
    blk = 400
    grid = N // blk
    return pl.pallas_call(
        _combine_body,
        grid=(grid,),
        in_specs=[
            pl.BlockSpec((1, blk, ROW), lambda i: (0, i, 0)),
            pl.BlockSpec((1, blk, ROW), lambda i: (1, i, 0)),
            pl.BlockSpec((ROW, HD), lambda i: (0, 0)),
            pl.BlockSpec((ROW, HD), lambda i: (0, 0)),
            pl.BlockSpec((ROW, HD), lambda i: (0, 0)),
            pl.BlockSpec((ROW, HD), lambda i: (0, 0)),
        ],
        out_specs=pl.BlockSpec((blk, HD), lambda i: (i, 0)),
        out_shape=jax.ShapeDtypeStruct((N, HD), jnp.float32),
    )(part, part, s1a, s1b, s2a, s2b)


def _selectors():
    s1a = np.zeros((ROW, HD), np.float32)
    s1b = np.zeros((ROW, HD), np.float32)
    s2a = np.zeros((ROW, HD), np.float32)
    s2b = np.zeros((ROW, HD), np.float32)
    for hl in range(CH):
        for d in range(D):
            s1a[hl * D + d, hl * D + d] = 1.0
            s1b[hl * D + d, KC + hl * D + d] = 1.0
            s2a[KC + hl, hl * D + d] = 1.0
            s2b[KC + hl, KC + hl * D + d] = 1.0
    return s1a, s1b, s2a, s2b


_S1A, _S1B, _S2A, _S2B = _selectors()


def kernel(h, edge_index, Wq, bq, Wk, bk, Wv, bv):
    wkv = jnp.concatenate([Wk[:, :KC], Wv[:, :KC], Wk[:, KC:], Wv[:, KC:]], axis=1)
    bkv = jnp.concatenate([bk[:KC], bv[:KC], bk[KC:], bv[KC:]])[None, :]
    q, kvp = _projections(h, Wq, wkv, bq[None, :], bkv)
    kv2 = jnp.concatenate([kvp[:, :2 * KC], kvp[:, 2 * KC:]], axis=0)
    q2 = jnp.concatenate([q[:, :KC], q[:, KC:]], axis=0)
    part = _sc_edge_phase(kv2, q2, edge_index[0], edge_index[1])
    return _combine(part, jnp.asarray(_S1A), jnp.asarray(_S1B),
                    jnp.asarray(_S2A), jnp.asarray(_S2B))


# speedup vs baseline: 6.1280x; 1.4062x over previous
import functools

import jax
import jax.numpy as jnp
import numpy as np
from jax import lax
from jax.experimental import pallas as pl
from jax.experimental.pallas import tpu as pltpu
from jax.experimental.pallas import tpu_sc as plsc

N = 10000
E = 320000
IN_DIM = 128
H = 8
D = 16
HD = H * D

NC = 2
NS = 16
CH = H // NC
KC = CH * D
EPT = E // NS
C = 80
NCHUNK = EPT // C
ROW = 80
N_PAD = 10240
RPS = N_PAD // NS

INV_SQRT_D = 1.0 / np.sqrt(D)


def _proj_body(h_ref, wq_ref, wkv_ref, bq_ref, bkv_ref, q_ref, kv_ref):
    hb = h_ref[...]
    q_ref[...] = jnp.dot(hb, wq_ref[...], preferred_element_type=jnp.float32) + bq_ref[...]
    kv_ref[...] = jnp.dot(hb, wkv_ref[...], preferred_element_type=jnp.float32) + bkv_ref[...]


def _projections(h, wq, wkv, bq, bkv):
    blk = 400
    grid = N // blk
    return pl.pallas_call(
        _proj_body,
        grid=(grid,),
        in_specs=[
            pl.BlockSpec((blk, IN_DIM), lambda i: (i, 0)),
            pl.BlockSpec((IN_DIM, HD), lambda i: (0, 0)),
            pl.BlockSpec((IN_DIM, 2 * HD), lambda i: (0, 0)),
            pl.BlockSpec((1, HD), lambda i: (0, 0)),
            pl.BlockSpec((1, 2 * HD), lambda i: (0, 0)),
        ],
        out_specs=[
            pl.BlockSpec((blk, HD), lambda i: (i, 0)),
            pl.BlockSpec((blk, 2 * HD), lambda i: (i, 0)),
        ],
        out_shape=[
            jax.ShapeDtypeStruct((N, HD), jnp.float32),
            jax.ShapeDtypeStruct((N, 2 * HD), jnp.float32),
        ],
    )(h, wq, wkv, bq, bkv)


def _sc_body(kv_hbm, q_hbm, src_hbm, dst_hbm, part_hbm,
             src_all, dst_all, dst0, dst1, gs0, gs1, gd0, gd1, kv0, kv1, q0, q1,
             m_v, zero_v, acc, semg0, semg1):
    c = lax.axis_index("c")
    s = lax.axis_index("s")
    coff = c * N
    dst_b = (dst0, dst1)
    gs_b = (gs0, gs1)
    gd_b = (gd0, gd1)
    kv_b = (kv0, kv1)
    q_b = (q0, q1)
    sem_b = (semg0, semg1)

    def zrow(i, _):
        for j in range(ROW // 16):
            zero_v[i, pl.ds(j * 16, 16)] = jnp.zeros((16,), jnp.float32)
        return 0
    lax.fori_loop(0, 8, zrow, 0)

    def zcopy(k, _):
        pltpu.sync_copy(zero_v, acc.at[pl.ds(pl.multiple_of(s * RPS + k * 8, 8), 8)])
        return 0
    lax.fori_loop(0, RPS // 8, zcopy, 0)

    pltpu.sync_copy(src_hbm.at[pl.ds(pl.multiple_of(s * EPT, 8), EPT)], src_all)
    pltpu.sync_copy(dst_hbm.at[pl.ds(pl.multiple_of(s * EPT, 8), EPT)], dst_all)
    plsc.subcore_barrier()

    lane = lax.broadcasted_iota(jnp.int32, (16,), 0)

    def prefetch(ci, t):
        base = ci * C
        for j in range(C // 16):
            sl = pl.ds(j * 16, 16)
            dv = dst_all[pl.ds(base + j * 16, 16)]
            dst_b[t][sl] = dv
            gs_b[t][sl] = src_all[pl.ds(base + j * 16, 16)] + coff
            gd_b[t][sl] = dv + coff
        pltpu.async_copy(kv_hbm.at[gs_b[t]], kv_b[t], sem_b[t])
        pltpu.async_copy(q_hbm.at[gd_b[t]], q_b[t], sem_b[t])

    def process(t):
        pltpu.make_async_copy(kv_hbm.at[gs_b[t]], kv_b[t], sem_b[t]).wait()
        pltpu.make_async_copy(q_hbm.at[gd_b[t]], q_b[t], sem_b[t]).wait()
        kv_v = kv_b[t]
        q_v = q_b[t]

        @plsc.parallel_loop(0, C, 1, unroll=2)
        def edge(e):
            zacc = jnp.zeros((16,), jnp.float32)
            for hh in range(CH):
                kvec = kv_v[e, pl.ds(hh * D, D)]
                qvec = q_v[e, pl.ds(hh * D, D)]
                sca = jnp.sum(kvec * qvec) * INV_SQRT_D
                ex = jnp.exp(jnp.clip(jnp.full((16,), sca), -5.0, 5.0))
                m_v[e, pl.ds(hh * D, D)] = kv_v[e, pl.ds(KC + hh * D, D)] * ex
                zacc = zacc + jnp.where(lane == hh, ex, 0.0)
            m_v[e, pl.ds(KC, 16)] = zacc

        pltpu.sync_copy(m_v, acc.at[dst_b[t]], add=True)

    prefetch(0, 0)

    def pair(i2, _):
        prefetch(2 * i2 + 1, 1)
        process(0)
        prefetch(2 * i2 + 2, 0)
        process(1)
        return 0

    lax.fori_loop(0, NCHUNK // 2 - 1, pair, 0)
    prefetch(NCHUNK - 1, 1)
    process(0)
    process(1)
    plsc.subcore_barrier()

    row0 = pl.multiple_of(s * RPS, 8)
    pltpu.sync_copy(acc.at[pl.ds(row0, RPS)], part_hbm.at[c, pl.ds(row0, RPS)])


def _sc_edge_phase(kv2, q2, src, dst):
    mesh = plsc.VectorSubcoreMesh(core_axis_name="c", subcore_axis_name="s",
                                  num_cores=NC, num_subcores=NS)
    f = pl.kernel(
        _sc_body,
        out_type=jax.ShapeDtypeStruct((NC, N_PAD, ROW), jnp.float32),
        mesh=mesh,
        compiler_params=pltpu.CompilerParams(needs_layout_passes=False,
                                             use_tc_tiling_on_sc=False),
        scratch_types=[
            pltpu.VMEM((EPT,), jnp.int32),
            pltpu.VMEM((EPT,), jnp.int32),
            pltpu.VMEM((C,), jnp.int32),
            pltpu.VMEM((C,), jnp.int32),
            pltpu.VMEM((C,), jnp.int32),
            pltpu.VMEM((C,), jnp.int32),
            pltpu.VMEM((C,), jnp.int32),
            pltpu.VMEM((C,), jnp.int32),
            pltpu.VMEM((C, 2 * KC), jnp.float32),
            pltpu.VMEM((C, 2 * KC), jnp.float32),
            pltpu.VMEM((C, KC), jnp.float32),
            pltpu.VMEM((C, KC), jnp.float32),
            pltpu.VMEM((C, ROW), jnp.float32),
            pltpu.VMEM((8, ROW), jnp.float32),
            pltpu.VMEM_SHARED((N_PAD, ROW), jnp.float32),
            pltpu.SemaphoreType.DMA,
            pltpu.SemaphoreType.DMA,
        ],
    )
    return f(kv2, q2, src, dst)


def _combine_body(p0_ref, p1_ref, s1a_ref, s1b_ref, s2a_ref, s2b_ref, o_ref):
    p0 = p0_ref[0]
    p1 = p1_ref[0]
    wv = (jnp.dot(p0, s1a_ref[...], preferred_element_type=jnp.float32)
          + jnp.dot(p1, s1b_ref[...], preferred_element_type=jnp.float32))
    z = (jnp.dot(p0, s2a_ref[...], preferred_element_type=jnp.float32)
         + jnp.dot(p1, s2b_ref[...], preferred_element_type=jnp.float32))
    o_ref[...] = wv / (z + 1e-6)


def _combine(part, s1a, s1b, s2a, s2b):
    blk = 400
    grid = N // blk
    return pl.pallas_call(
        _combine_body,
        grid=(grid,),
        in_specs=[
            pl.BlockSpec((1, blk, ROW), lambda i: (0, i, 0)),
            pl.BlockSpec((1, blk, ROW), lambda i: (1, i, 0)),
            pl.BlockSpec((ROW, HD), lambda i: (0, 0)),
            pl.BlockSpec((ROW, HD), lambda i: (0, 0)),
            pl.BlockSpec((ROW, HD), lambda i: (0, 0)),
            pl.BlockSpec((ROW, HD), lambda i: (0, 0)),
        ],
        out_specs=pl.BlockSpec((blk, HD), lambda i: (i, 0)),
        out_shape=jax.ShapeDtypeStruct((N, HD), jnp.float32),
    )(part, part, s1a, s1b, s2a, s2b)


def _selectors():
    s1a = np.zeros((ROW, HD), np.float32)
    s1b = np.zeros((ROW, HD), np.float32)
    s2a = np.zeros((ROW, HD), np.float32)
    s2b = np.zeros((ROW, HD), np.float32)
    for hl in range(CH):
        for d in range(D):
            s1a[hl * D + d, hl * D + d] = 1.0
            s1b[hl * D + d, KC + hl * D + d] = 1.0
            s2a[KC + hl, hl * D + d] = 1.0
            s2b[KC + hl, KC + hl * D + d] = 1.0
    return s1a, s1b, s2a, s2b


_S1A, _S1B, _S2A, _S2B = _selectors()


def kernel(h, edge_index, Wq, bq, Wk, bk, Wv, bv):
    wkv = jnp.concatenate([Wk[:, :KC], Wv[:, :KC], Wk[:, KC:], Wv[:, KC:]], axis=1)
    bkv = jnp.concatenate([bk[:KC], bv[:KC], bk[KC:], bv[KC:]])[None, :]
    q, kvp = _projections(h, Wq, wkv, bq[None, :], bkv)
    kv2 = jnp.concatenate([kvp[:, :2 * KC], kvp[:, 2 * KC:]], axis=0)
    q2 = jnp.concatenate([q[:, :KC], q[:, KC:]], axis=0)
    part = _sc_edge_phase(kv2, q2, edge_index[0], edge_index[1])
    return _combine(part, jnp.asarray(_S1A), jnp.asarray(_S1B),
                    jnp.asarray(_S2A), jnp.asarray(_S2B))


# speedup vs baseline: 6.1833x; 1.0090x over previous
import functools

import jax
import jax.numpy as jnp
import numpy as np
from jax import lax
from jax.experimental import pallas as pl
from jax.experimental.pallas import tpu as pltpu
from jax.experimental.pallas import tpu_sc as plsc

N = 10000
E = 320000
IN_DIM = 128
H = 8
D = 16
HD = H * D

NC = 2
NS = 16
CH = H // NC
KC = CH * D
EPT = E // NS
C = 80
NCHUNK = EPT // C
ROW = 80
N_PAD = 10240
RPS = N_PAD // NS

INV_SQRT_D = 1.0 / np.sqrt(D)


def _proj_body(h_ref, wq_ref, wkv_ref, bq_ref, bkv_ref, q_ref, kv_ref):
    hb = h_ref[...]
    q_ref[...] = jnp.dot(hb, wq_ref[...], preferred_element_type=jnp.float32) + bq_ref[...]
    kv_ref[...] = jnp.dot(hb, wkv_ref[...], preferred_element_type=jnp.float32) + bkv_ref[...]


def _projections(h, wq, wkv, bq, bkv):
    blk = 400
    grid = N // blk
    return pl.pallas_call(
        _proj_body,
        grid=(grid,),
        in_specs=[
            pl.BlockSpec((blk, IN_DIM), lambda i: (i, 0)),
            pl.BlockSpec((IN_DIM, HD), lambda i: (0, 0)),
            pl.BlockSpec((IN_DIM, 2 * HD), lambda i: (0, 0)),
            pl.BlockSpec((1, HD), lambda i: (0, 0)),
            pl.BlockSpec((1, 2 * HD), lambda i: (0, 0)),
        ],
        out_specs=[
            pl.BlockSpec((blk, HD), lambda i: (i, 0)),
            pl.BlockSpec((blk, 2 * HD), lambda i: (i, 0)),
        ],
        out_shape=[
            jax.ShapeDtypeStruct((N, HD), jnp.float32),
            jax.ShapeDtypeStruct((N, 2 * HD), jnp.float32),
        ],
    )(h, wq, wkv, bq, bkv)


def _sc_body(kv_hbm, q_hbm, src_hbm, dst_hbm, zeros_hbm, part_hbm,
             src_all, dst_all, dst0, dst1, gs0, gs1, gd0, gd1, kv0, kv1, q0, q1,
             m_v, acc, semg0, semg1):
    c = lax.axis_index("c")
    s = lax.axis_index("s")
    coff = c * N
    dst_b = (dst0, dst1)
    gs_b = (gs0, gs1)
    gd_b = (gd0, gd1)
    kv_b = (kv0, kv1)
    q_b = (q0, q1)
    sem_b = (semg0, semg1)

    zr = pl.multiple_of(s * RPS, 8)
    pltpu.sync_copy(zeros_hbm.at[pl.ds(zr, RPS)], acc.at[pl.ds(zr, RPS)])

    pltpu.sync_copy(src_hbm.at[pl.ds(pl.multiple_of(s * EPT, 8), EPT)], src_all)
    pltpu.sync_copy(dst_hbm.at[pl.ds(pl.multiple_of(s * EPT, 8), EPT)], dst_all)
    plsc.subcore_barrier()

    lane = lax.broadcasted_iota(jnp.int32, (16,), 0)

    def prefetch(ci, t):
        base = ci * C
        for j in range(C // 16):
            sl = pl.ds(j * 16, 16)
            dv = dst_all[pl.ds(base + j * 16, 16)]
            dst_b[t][sl] = dv
            gs_b[t][sl] = src_all[pl.ds(base + j * 16, 16)] + coff
            gd_b[t][sl] = dv + coff
        pltpu.async_copy(kv_hbm.at[gs_b[t]], kv_b[t], sem_b[t])
        pltpu.async_copy(q_hbm.at[gd_b[t]], q_b[t], sem_b[t])

    def process(t):
        pltpu.make_async_copy(kv_hbm.at[gs_b[t]], kv_b[t], sem_b[t]).wait()
        pltpu.make_async_copy(q_hbm.at[gd_b[t]], q_b[t], sem_b[t]).wait()
        kv_v = kv_b[t]
        q_v = q_b[t]

        @plsc.parallel_loop(0, C, 1, unroll=4)
        def edge(e):
            zacc = jnp.zeros((16,), jnp.float32)
            for hh in range(CH):
                kvec = kv_v[e, pl.ds(hh * D, D)]
                qvec = q_v[e, pl.ds(hh * D, D)]
                sca = jnp.sum(kvec * qvec) * INV_SQRT_D
                ex = jnp.exp(jnp.clip(jnp.full((16,), sca), -5.0, 5.0))
                m_v[e, pl.ds(hh * D, D)] = kv_v[e, pl.ds(KC + hh * D, D)] * ex
                zacc = zacc + jnp.where(lane == hh, ex, 0.0)
            m_v[e, pl.ds(KC, 16)] = zacc

        pltpu.sync_copy(m_v, acc.at[dst_b[t]], add=True)

    prefetch(0, 0)

    def pair(i2, _):
        prefetch(2 * i2 + 1, 1)
        process(0)
        prefetch(2 * i2 + 2, 0)
        process(1)
        return 0

    lax.fori_loop(0, NCHUNK // 2 - 1, pair, 0)
    prefetch(NCHUNK - 1, 1)
    process(0)
    process(1)
    plsc.subcore_barrier()

    row0 = pl.multiple_of(s * RPS, 8)
    pltpu.sync_copy(acc.at[pl.ds(row0, RPS)], part_hbm.at[c, pl.ds(row0, RPS)])


def _sc_edge_phase(kv2, q2, src, dst):
    mesh = plsc.VectorSubcoreMesh(core_axis_name="c", subcore_axis_name="s",
                                  num_cores=NC, num_subcores=NS)
    f = pl.kernel(
        _sc_body,
        out_type=jax.ShapeDtypeStruct((NC, N_PAD, ROW), jnp.float32),
        mesh=mesh,
        compiler_params=pltpu.CompilerParams(needs_layout_passes=False,
                                             use_tc_tiling_on_sc=False),
        scratch_types=[
            pltpu.VMEM((EPT,), jnp.int32),
            pltpu.VMEM((EPT,), jnp.int32),
            pltpu.VMEM((C,), jnp.int32),
            pltpu.VMEM((C,), jnp.int32),
            pltpu.VMEM((C,), jnp.int32),
            pltpu.VMEM((C,), jnp.int32),
            pltpu.VMEM((C,), jnp.int32),
            pltpu.VMEM((C,), jnp.int32),
            pltpu.VMEM((C, 2 * KC), jnp.float32),
            pltpu.VMEM((C, 2 * KC), jnp.float32),
            pltpu.VMEM((C, KC), jnp.float32),
            pltpu.VMEM((C, KC), jnp.float32),
            pltpu.VMEM((C, ROW), jnp.float32),
            pltpu.VMEM_SHARED((N_PAD, ROW), jnp.float32),
            pltpu.SemaphoreType.DMA,
            pltpu.SemaphoreType.DMA,
        ],
    )
    return f(kv2, q2, src, dst, jnp.zeros((N_PAD, ROW), jnp.float32))


def _combine_body(p0_ref, p1_ref, s1a_ref, s1b_ref, s2a_ref, s2b_ref, o_ref):
    p0 = p0_ref[0]
    p1 = p1_ref[0]
    wv = (jnp.dot(p0, s1a_ref[...], preferred_element_type=jnp.float32)
          + jnp.dot(p1, s1b_ref[...], preferred_element_type=jnp.float32))
    z = (jnp.dot(p0, s2a_ref[...], preferred_element_type=jnp.float32)
         + jnp.dot(p1, s2b_ref[...], preferred_element_type=jnp.float32))
    o_ref[...] = wv / (z + 1e-6)


def _combine(part, s1a, s1b, s2a, s2b):
    blk = 400
    grid = N // blk
    return pl.pallas_call(
        _combine_body,
        grid=(grid,),
        in_specs=[
            pl.BlockSpec((1, blk, ROW), lambda i: (0, i, 0)),
            pl.BlockSpec((1, blk, ROW), lambda i: (1, i, 0)),
            pl.BlockSpec((ROW, HD), lambda i: (0, 0)),
            pl.BlockSpec((ROW, HD), lambda i: (0, 0)),
            pl.BlockSpec((ROW, HD), lambda i: (0, 0)),
            pl.BlockSpec((ROW, HD), lambda i: (0, 0)),
        ],
        out_specs=pl.BlockSpec((blk, HD), lambda i: (i, 0)),
        out_shape=jax.ShapeDtypeStruct((N, HD), jnp.float32),
    )(part, part, s1a, s1b, s2a, s2b)


def _selectors():
    s1a = np.zeros((ROW, HD), np.float32)
    s1b = np.zeros((ROW, HD), np.float32)
    s2a = np.zeros((ROW, HD), np.float32)
    s2b = np.zeros((ROW, HD), np.float32)
    for hl in range(CH):
        for d in range(D):
            s1a[hl * D + d, hl * D + d] = 1.0
            s1b[hl * D + d, KC + hl * D + d] = 1.0
            s2a[KC + hl, hl * D + d] = 1.0
            s2b[KC + hl, KC + hl * D + d] = 1.0
    return s1a, s1b, s2a, s2b


_S1A, _S1B, _S2A, _S2B = _selectors()


def kernel(h, edge_index, Wq, bq, Wk, bk, Wv, bv):
    wkv = jnp.concatenate([Wk[:, :KC], Wv[:, :KC], Wk[:, KC:], Wv[:, KC:]], axis=1)
    bkv = jnp.concatenate([bk[:KC], bv[:KC], bk[KC:], bv[KC:]])[None, :]
    q, kvp = _projections(h, Wq, wkv, bq[None, :], bkv)
    kv2 = jnp.concatenate([kvp[:, :2 * KC], kvp[:, 2 * KC:]], axis=0)
    q2 = jnp.concatenate([q[:, :KC], q[:, KC:]], axis=0)
    part = _sc_edge_phase(kv2, q2, edge_index[0], edge_index[1])
    return _combine(part, jnp.asarray(_S1A), jnp.asarray(_S1B),
                    jnp.asarray(_S2A), jnp.asarray(_S2B))
